# baseline (speedup 1.0000x reference)
import jax
import jax.numpy as jnp
from jax import lax
from jax.experimental import pallas as pl
from jax.experimental.pallas import tpu as pltpu
from jax.experimental.pallas import tpu_sc as plsc

SEQ = 2048
DIM = 128
NB = 4

_info = plsc.get_sparse_core_info()
_NC = _info.num_cores
_NS = _info.num_subcores
_L = _info.num_lanes
NW = _NC * _NS
SPW = SEQ // NW
BPW = NB * SPW


def _sc_body(idx_hbm, tok_hbm, pos_hbm, out_hbm, idx_v, rows_v, pos_v,
             gsems, psem, osem):
    wid = lax.axis_index("s") * _NC + lax.axis_index("c")
    s0 = wid * SPW

    icopies = [
        pltpu.async_copy(idx_hbm.at[b, pl.ds(s0, SPW)], idx_v.at[b], psem)
        for b in range(NB)
    ]
    for ic in icopies:
        ic.wait()
    gathers = []
    pcopy = None
    for b in range(NB):
        gathers.append(
            pltpu.async_copy(tok_hbm.at[idx_v.at[b]],
                             rows_v.at[pl.ds(b * SPW, SPW)], gsems.at[b])
        )
        if b == 0:
            pcopy = pltpu.async_copy(pos_hbm.at[pl.ds(s0, SPW)], pos_v, psem)

    outs = []
    for b in range(NB):
        with jax.named_scope(f"gwait{b}"):
            gathers[b].wait()
            if b == 0:
                pcopy.wait()
        r0 = b * SPW

        def add_row(i, carry):
            for c in range(DIM // _L):
                sl = pl.ds(c * _L, _L)
                rows_v[r0 + i, sl] = rows_v[r0 + i, sl] + pos_v[i, sl]
            return carry

        with jax.named_scope(f"add{b}"):
            lax.fori_loop(0, SPW, add_row, 0)
        outs.append(
            pltpu.async_copy(rows_v.at[pl.ds(r0, SPW)],
                             out_hbm.at[pl.ds(b * SEQ + s0, SPW)], osem)
        )
    with jax.named_scope("owait"):
        for o in outs:
            o.wait()


@jax.jit
def _sc_embed(idx, token_table, pos_table):
    kern = pl.kernel(
        _sc_body,
        out_type=jax.ShapeDtypeStruct((NB * SEQ, DIM), jnp.float32),
        mesh=plsc.VectorSubcoreMesh(core_axis_name="c", subcore_axis_name="s"),
        scratch_types=[
            pltpu.VMEM((NB, SPW), jnp.int32),
            pltpu.VMEM((BPW, DIM), jnp.float32),
            pltpu.VMEM((SPW, DIM), jnp.float32),
            pltpu.SemaphoreType.DMA((NB,)),
            pltpu.SemaphoreType.DMA,
            pltpu.SemaphoreType.DMA,
        ],
    )
    return kern(idx, token_table, pos_table)


def kernel(inputs, token_table, pos_table):
    out = _sc_embed(inputs.astype(jnp.int32), token_table, pos_table)
    return out.reshape(NB, SEQ, DIM)


# speedup vs baseline: 1.0169x; 1.0169x over previous
import jax
import jax.numpy as jnp
from jax import lax
from jax.experimental import pallas as pl
from jax.experimental.pallas import tpu as pltpu
from jax.experimental.pallas import tpu_sc as plsc

SEQ = 2048
DIM = 128
NB = 4

_info = plsc.get_sparse_core_info()
_NC = _info.num_cores
_NS = _info.num_subcores
_L = _info.num_lanes
NW = _NC * _NS
SPW = SEQ // NW
BPW = NB * SPW


def _sc_body(idx_hbm, tok_hbm, pos_hbm, out_hbm, idx_v, rows_v, pos_v,
             gsems, psem, osem):
    wid = lax.axis_index("s") * _NC + lax.axis_index("c")
    s0 = wid * SPW

    icopies = [
        pltpu.async_copy(idx_hbm.at[b, pl.ds(s0, SPW)], idx_v.at[b], psem)
        for b in range(NB)
    ]
    for ic in icopies:
        ic.wait()
    gathers = []
    pcopy = None
    for b in range(NB):
        gathers.append(
            pltpu.async_copy(tok_hbm.at[idx_v.at[b]],
                             rows_v.at[pl.ds(b * SPW, SPW)], gsems.at[b])
        )
        if b == 0:
            pcopy = pltpu.async_copy(pos_hbm.at[pl.ds(s0, SPW)], pos_v, psem)

    with jax.named_scope("gwait"):
        for g in gathers:
            g.wait()
        pcopy.wait()

    def add_row(i, carry):
        for c in range(DIM // _L):
            sl = pl.ds(c * _L, _L)
            p = pos_v[i, sl]
            for b in range(NB):
                r = b * SPW + i
                rows_v[r, sl] = rows_v[r, sl] + p
        return carry

    QR = SPW // 4
    outs = []
    for q in range(4):
        with jax.named_scope(f"add{q}"):
            lax.fori_loop(q * QR, (q + 1) * QR, add_row, 0)
        for b in range(NB):
            outs.append(
                pltpu.async_copy(
                    rows_v.at[pl.ds(b * SPW + q * QR, QR)],
                    out_hbm.at[pl.ds(b * SEQ + s0 + q * QR, QR)], osem)
            )
    with jax.named_scope("owait"):
        for o in outs:
            o.wait()


@jax.jit
def _sc_embed(idx, token_table, pos_table):
    kern = pl.kernel(
        _sc_body,
        out_type=jax.ShapeDtypeStruct((NB * SEQ, DIM), jnp.float32),
        mesh=plsc.VectorSubcoreMesh(core_axis_name="c", subcore_axis_name="s"),
        scratch_types=[
            pltpu.VMEM((NB, SPW), jnp.int32),
            pltpu.VMEM((BPW, DIM), jnp.float32),
            pltpu.VMEM((SPW, DIM), jnp.float32),
            pltpu.SemaphoreType.DMA((NB,)),
            pltpu.SemaphoreType.DMA,
            pltpu.SemaphoreType.DMA,
        ],
    )
    return kern(idx, token_table, pos_table)


def kernel(inputs, token_table, pos_table):
    out = _sc_embed(inputs.astype(jnp.int32), token_table, pos_table)
    return out.reshape(NB, SEQ, DIM)


# speedup vs baseline: 1.0337x; 1.0165x over previous
import jax
import jax.numpy as jnp
from jax import lax
from jax.experimental import pallas as pl
from jax.experimental.pallas import tpu as pltpu
from jax.experimental.pallas import tpu_sc as plsc

SEQ = 2048
DIM = 128
NB = 4

_info = plsc.get_sparse_core_info()
_NC = _info.num_cores
_NS = _info.num_subcores
_L = _info.num_lanes
NW = _NC * _NS
SPW = SEQ // NW
BPW = NB * SPW


def _sc_body(idx_hbm, tok_hbm, pos_hbm, out_hbm, idx_v, rows_v, pos_v,
             gsems, psem, isem, osem):
    wid = lax.axis_index("s") * _NC + lax.axis_index("c")
    s0 = wid * SPW

    pcopy = pltpu.async_copy(pos_hbm.at[pl.ds(s0, SPW)], pos_v, psem)
    icopies = [
        pltpu.async_copy(idx_hbm.at[b, pl.ds(s0, SPW)],
                         idx_v.at[b // 2, pl.ds((b % 2) * SPW, SPW)], isem)
        for b in range(NB)
    ]
    for ic in icopies:
        ic.wait()
    gathers = [
        pltpu.async_copy(tok_hbm.at[idx_v.at[g]],
                         rows_v.at[pl.ds(g * 2 * SPW, 2 * SPW)], gsems.at[g])
        for g in range(NB // 2)
    ]

    with jax.named_scope("gwait"):
        for g in gathers:
            g.wait()
        pcopy.wait()

    def add_row(i, carry):
        for c in range(DIM // _L):
            sl = pl.ds(c * _L, _L)
            p = pos_v[i, sl]
            for b in range(NB):
                r = b * SPW + i
                rows_v[r, sl] = rows_v[r, sl] + p
        return carry

    QR = SPW // 4
    outs = []
    for q in range(4):
        with jax.named_scope(f"add{q}"):
            lax.fori_loop(q * QR, (q + 1) * QR, add_row, 0)
        for b in range(NB):
            outs.append(
                pltpu.async_copy(
                    rows_v.at[pl.ds(b * SPW + q * QR, QR)],
                    out_hbm.at[pl.ds(b * SEQ + s0 + q * QR, QR)], osem)
            )
    with jax.named_scope("owait"):
        for o in outs:
            o.wait()


@jax.jit
def _sc_embed(idx, token_table, pos_table):
    kern = pl.kernel(
        _sc_body,
        out_type=jax.ShapeDtypeStruct((NB * SEQ, DIM), jnp.float32),
        mesh=plsc.VectorSubcoreMesh(core_axis_name="c", subcore_axis_name="s"),
        scratch_types=[
            pltpu.VMEM((NB // 2, 2 * SPW), jnp.int32),
            pltpu.VMEM((BPW, DIM), jnp.float32),
            pltpu.VMEM((SPW, DIM), jnp.float32),
            pltpu.SemaphoreType.DMA((NB // 2,)),
            pltpu.SemaphoreType.DMA,
            pltpu.SemaphoreType.DMA,
            pltpu.SemaphoreType.DMA,
        ],
    )
    return kern(idx, token_table, pos_table)


def kernel(inputs, token_table, pos_table):
    out = _sc_embed(inputs.astype(jnp.int32), token_table, pos_table)
    return out.reshape(NB, SEQ, DIM)
